# Initial kernel scaffold; baseline (speedup 1.0000x reference)
import jax
import jax.numpy as jnp
from jax.experimental import pallas as pl


def kernel(keys, queue):
    raise NotImplementedError("write your pallas kernel here")


# speedup vs baseline: 1.0696x; 1.0696x over previous
import jax
import jax.numpy as jnp
from jax.experimental import pallas as pl

FEATURE = 1024
QUEUE = 65536
BATCH = 4096
BLOCK_C = 512
N_KEY_BLOCKS = BATCH // BLOCK_C
N_BLOCKS = QUEUE // BLOCK_C


def _body(keys_ref, queue_ref, out_ref):
    j = pl.program_id(0)

    @pl.when(j < N_KEY_BLOCKS)
    def _():
        out_ref[...] = keys_ref[...].T

    @pl.when(j >= N_KEY_BLOCKS)
    def _():
        out_ref[...] = queue_ref[...]


def kernel(keys, queue):
    return pl.pallas_call(
        _body,
        grid=(N_BLOCKS,),
        in_specs=[
            pl.BlockSpec((BLOCK_C, FEATURE),
                         lambda j: (jnp.minimum(j, N_KEY_BLOCKS - 1), 0)),
            pl.BlockSpec((FEATURE, BLOCK_C), lambda j: (0, j)),
        ],
        out_specs=pl.BlockSpec((FEATURE, BLOCK_C), lambda j: (0, j)),
        out_shape=jax.ShapeDtypeStruct((FEATURE, QUEUE), jnp.float32),
    )(keys, queue)


# speedup vs baseline: 1.2151x; 1.1361x over previous
import jax
import jax.numpy as jnp
from jax.experimental import pallas as pl

FEATURE = 1024
QUEUE = 65536
BATCH = 4096
BLOCK_C = 2048
N_KEY_BLOCKS = BATCH // BLOCK_C
N_BLOCKS = QUEUE // BLOCK_C


def _body(keys_ref, queue_ref, out_ref):
    j = pl.program_id(0)

    @pl.when(j < N_KEY_BLOCKS)
    def _():
        out_ref[...] = keys_ref[...].T

    @pl.when(j >= N_KEY_BLOCKS)
    def _():
        out_ref[...] = queue_ref[...]


def kernel(keys, queue):
    return pl.pallas_call(
        _body,
        grid=(N_BLOCKS,),
        in_specs=[
            pl.BlockSpec((BLOCK_C, FEATURE),
                         lambda j: (jnp.minimum(j, N_KEY_BLOCKS - 1), 0)),
            pl.BlockSpec((FEATURE, BLOCK_C),
                         lambda j: (0, jnp.maximum(j, N_KEY_BLOCKS))),
        ],
        out_specs=pl.BlockSpec((FEATURE, BLOCK_C), lambda j: (0, j)),
        out_shape=jax.ShapeDtypeStruct((FEATURE, QUEUE), jnp.float32),
    )(keys, queue)
